# baseline (speedup 1.0000x reference)
import functools

import jax
import jax.numpy as jnp
from jax import lax
from jax.experimental import pallas as pl
from jax.experimental.pallas import tpu as pltpu
from jax.experimental.pallas import tpu_sc as plsc

B = 16384
D = 16
OUTDIM = 64
EPS = 1e-5

NUM_CORES = 2
NUM_SUBCORES = 16
NW = NUM_CORES * NUM_SUBCORES
BPW = B // NW


def _sc_gather_body(emb_hbm, alpha_hbm, idx_hbm, e_out, a_out,
                    idx_v, rows_v, arow_v, sem_e, sem_a):
    wid = lax.axis_index("s") * NUM_CORES + lax.axis_index("c")
    base = wid * BPW
    pltpu.sync_copy(idx_hbm.at[pl.ds(base, BPW)], idx_v)
    cp_e = pltpu.async_copy(emb_hbm.at[idx_v], rows_v, sem_e)
    cp_a = pltpu.async_copy(alpha_hbm.at[idx_v], arow_v, sem_a)
    cp_e.wait()
    pltpu.sync_copy(rows_v, e_out.at[pl.ds(base, BPW)])
    cp_a.wait()
    pltpu.sync_copy(arow_v, a_out.at[pl.ds(base, BPW)])


@functools.lru_cache(maxsize=None)
def _sc_gather():
    return pl.kernel(
        _sc_gather_body,
        out_type=(
            jax.ShapeDtypeStruct((B, D), jnp.float32),
            jax.ShapeDtypeStruct((B,), jnp.float32),
        ),
        mesh=plsc.VectorSubcoreMesh(
            core_axis_name="c", subcore_axis_name="s",
            num_cores=NUM_CORES, num_subcores=NUM_SUBCORES,
        ),
        scratch_types=[
            pltpu.VMEM((BPW,), jnp.int32),
            pltpu.VMEM((BPW, D), jnp.float32),
            pltpu.VMEM((BPW,), jnp.float32),
            pltpu.SemaphoreType.DMA,
            pltpu.SemaphoreType.DMA,
        ],
        compiler_params=pltpu.CompilerParams(use_tc_tiling_on_sc=False),
    )


def _tc_body(e_ref, wt_ref, g_ref, bb_ref, out_ref):
    e = e_ref[...]
    wt = wt_ref[...]
    inv_b = 1.0 / B
    mean_e = jnp.sum(e, axis=0, keepdims=True) * inv_b
    smom = lax.dot_general(e, e, (((0,), (0,)), ((), ())),
                           preferred_element_type=jnp.float32) * inv_b
    m = lax.dot_general(mean_e, wt, (((1,), (0,)), ((), ())))
    p = lax.dot_general(smom, wt, (((1,), (0,)), ((), ())))
    ey2 = jnp.sum(wt * p, axis=0, keepdims=True)
    var = ey2 - m * m
    scale = g_ref[...] * lax.rsqrt(var + EPS)
    shift = bb_ref[...] - m * scale
    wp = wt * scale
    out_ref[...] = lax.dot_general(
        e, wp, (((1,), (0,)), ((), ())),
        preferred_element_type=jnp.float32) + shift


_tc_transform = pl.pallas_call(
    _tc_body,
    out_shape=jax.ShapeDtypeStruct((B, OUTDIM), jnp.float32),
)


@jax.jit
def kernel(x, emb_table, alpha_table, W, b, gamma, beta):
    del b
    e, alpha = _sc_gather()(emb_table, alpha_table.reshape(-1), x)
    y_bn = _tc_transform(e, W.T, gamma.reshape(1, OUTDIM),
                         beta.reshape(1, OUTDIM))
    return (y_bn, alpha.reshape(B, 1))


# speedup vs baseline: 3.3702x; 3.3702x over previous
import functools

import jax
import jax.numpy as jnp
from jax import lax
from jax.experimental import pallas as pl
from jax.experimental.pallas import tpu as pltpu
from jax.experimental.pallas import tpu_sc as plsc

B = 16384
D = 16
OUTDIM = 64
NE = 1000000
EPS = 1e-5

NUM_CORES = 2
NUM_SUBCORES = 16
NW = NUM_CORES * NUM_SUBCORES
BPW = B // NW


TBLK = 131072
NTBLK = (NE + TBLK - 1) // TBLK
FSTRIDE = NTBLK * TBLK


def _detile_body(emb_ref, flat_ref):
    sid = pl.program_id(2)
    for s in range(8):
        @pl.when(sid == s)
        def _():
            flat_ref[...] = emb_ref[s, :]


_tc_detile_tab = pl.pallas_call(
    _detile_body,
    grid=(2, NTBLK, 8),
    in_specs=[pl.BlockSpec((8, TBLK), lambda t, j, s: (t, j))],
    out_specs=pl.BlockSpec(
        (TBLK,), lambda t, j, s: ((t * 8 + s) * NTBLK + j,)),
    out_shape=jax.ShapeDtypeStruct((D * FSTRIDE,), jnp.float32),
)

ABLK = 131072
NABLK = (NE + ABLK - 1) // ABLK


def _alpha_body(alpha_ref, alin_ref):
    alin_ref[...] = alpha_ref[0, :]


_tc_detile_alpha = pl.pallas_call(
    _alpha_body,
    grid=(NABLK,),
    in_specs=[pl.BlockSpec((1, ABLK), lambda j: (0, j))],
    out_specs=pl.BlockSpec((ABLK,), lambda j: (j,)),
    out_shape=jax.ShapeDtypeStruct((NABLK * ABLK,), jnp.float32),
)


def _sc_gather_body(tab_hbm, alpha_hbm, idx_hbm, e_out, a_out,
                    idx_v, idxf_v, e_loc, arow_v, sem_e, sem_a):
    wid = lax.axis_index("s") * NUM_CORES + lax.axis_index("c")
    base = wid * BPW
    pltpu.sync_copy(idx_hbm.at[pl.ds(base, BPW)], idx_v)
    kvec = lax.iota(jnp.int32, 16) * FSTRIDE

    def _expand(r, c):
        splat = plsc.load_gather(idx_v, [jnp.full((16,), 1, jnp.int32) * r])
        idxf_v[pl.ds(r * D, 16)] = splat + kvec
        return c

    lax.fori_loop(0, BPW, _expand, 0)
    cp_a = pltpu.async_copy(alpha_hbm.at[idx_v], arow_v, sem_a)
    cp_e = pltpu.async_copy(tab_hbm.at[idxf_v], e_loc, sem_e)
    cp_e.wait()
    pltpu.sync_copy(e_loc, e_out.at[pl.ds(base * D, BPW * D)])
    cp_a.wait()
    pltpu.sync_copy(arow_v, a_out.at[pl.ds(base, BPW)])


@functools.lru_cache(maxsize=None)
def _sc_gather():
    return pl.kernel(
        _sc_gather_body,
        out_type=(
            jax.ShapeDtypeStruct((B * D,), jnp.float32),
            jax.ShapeDtypeStruct((B,), jnp.float32),
        ),
        mesh=plsc.VectorSubcoreMesh(
            core_axis_name="c", subcore_axis_name="s",
            num_cores=NUM_CORES, num_subcores=NUM_SUBCORES,
        ),
        scratch_types=[
            pltpu.VMEM((BPW,), jnp.int32),
            pltpu.VMEM((BPW * D,), jnp.int32),
            pltpu.VMEM((BPW * D,), jnp.float32),
            pltpu.VMEM((BPW,), jnp.float32),
            pltpu.SemaphoreType.DMA,
            pltpu.SemaphoreType.DMA,
        ],
        compiler_params=pltpu.CompilerParams(use_tc_tiling_on_sc=False,
                                             needs_layout_passes=False),
    )


def _tc_body(e_ref, wt_ref, g_ref, bb_ref, out_ref):
    e = e_ref[...]
    wt = wt_ref[...]
    inv_b = 1.0 / B
    ones_b = jnp.ones((B, 1), jnp.float32)
    ones_d = jnp.ones((D, 1), jnp.float32)
    mean_e = lax.dot_general(e, ones_b, (((0,), (0,)), ((), ())),
                             preferred_element_type=jnp.float32) * inv_b
    smom = lax.dot_general(e, e, (((0,), (0,)), ((), ())),
                           preferred_element_type=jnp.float32) * inv_b
    m_t = lax.dot_general(wt, mean_e, (((0,), (0,)), ((), ())))
    p = lax.dot_general(smom, wt, (((1,), (0,)), ((), ())))
    ey2_t = lax.dot_general(wt * p, ones_d, (((0,), (0,)), ((), ())))
    var_t = ey2_t - m_t * m_t
    scale_t = g_ref[...] * lax.rsqrt(var_t + EPS)
    shift_t = bb_ref[...] - m_t * scale_t
    y_t = lax.dot_general(wt, e, (((0,), (1,)), ((), ())),
                          preferred_element_type=jnp.float32)
    out_ref[...] = y_t * scale_t + shift_t


_tc_transform = pl.pallas_call(
    _tc_body,
    out_shape=jax.ShapeDtypeStruct((OUTDIM, B), jnp.float32),
)


@jax.jit
def kernel(x, emb_table, alpha_table, W, b, gamma, beta):
    del b
    flat_tab = _tc_detile_tab(emb_table.T)
    alpha_lin = _tc_detile_alpha(alpha_table.T)
    e_flat, alpha = _sc_gather()(flat_tab, alpha_lin, x)
    e = e_flat.reshape(B, D)
    y_t = _tc_transform(e, W.T, gamma.reshape(OUTDIM, 1),
                        beta.reshape(OUTDIM, 1))
    return (y_t.T, alpha.reshape(B, 1))
